# baseline (speedup 1.0000x reference)
import jax
import jax.numpy as jnp
from jax.experimental import pallas as pl
from jax.experimental.pallas import tpu as pltpu

_N = 10000
_D = 128
_H = 64
_L = 2


def _level_bounds(n):
    levs = []
    start, size = 0, 1
    while start < n:
        levs.append((start, min(start + size, n)))
        start += size
        size *= 2
    return levs


_LEVELS = _level_bounds(_N)

_FWD_CHUNKS = []
for _s, _e in reversed(_LEVELS):
    _nl = min(max((_N - 2) // 2 + 1 - _s, 0), _e - _s)
    _nr = min(max((_N - 3) // 2 + 1 - _s, 0), _e - _s)
    if _nr > 0:
        _FWD_CHUNKS.append((_s, _s + _nr, 'both'))
    if _nl > _nr:
        _FWD_CHUNKS.append((_s + _nr, _s + _nl, 'left'))
    if _nl < _e - _s:
        _FWD_CHUNKS.append((_s + _nl, _e, 'none'))


def _sig(v):
    return 0.5 * jnp.tanh(0.5 * v) + 0.5


def _btree_kernel(feat_ref, wall_ref, wnone_ref, bnone_ref, bff_ref,
                  blf_ref, wxb_ref, whb_ref, bbb_ref, out_ref,
                  x1_ref, sf_ref, sb_ref):
    f32 = jnp.float32

    def fwd_chunk(l, x_ref, dst_ref, s, e, kind):
        m = e - s
        xs = x_ref[s:e, :]
        if kind == 'none':
            g = jnp.dot(xs, wnone_ref[l], preferred_element_type=f32)
            g = g + bnone_ref[l]
            sa = _sig(g[:, 0:2 * _H])
            cg = sa[:, 0:_H] * jnp.tanh(g[:, 2 * _H:3 * _H])
        elif kind == 'left':
            lch = sf_ref[pl.Slice(2 * s + 1, m, 2), :]
            g = jnp.dot(jnp.concatenate([xs, lch], axis=1),
                        wall_ref[l, 0:_D + 2 * _H, :],
                        preferred_element_type=f32)
            g = g + blf_ref[l]
            sa = _sig(g[:, 0:2 * _H])
            sb = _sig(g[:, 2 * _H:4 * _H])
            cg = (sa[:, 0:_H] * jnp.tanh(g[:, 4 * _H:5 * _H])
                  + sb[:, 0:_H] * lch[:, 0:_H])
        else:
            cs = 2 * s + 1
            lch = sf_ref[pl.Slice(cs, m, 2), :]
            rch = sf_ref[pl.Slice(cs + 1, m, 2), :]
            g = jnp.dot(jnp.concatenate([xs, lch, rch], axis=1), wall_ref[l],
                        preferred_element_type=f32)
            g = g + bff_ref[l]
            sa = _sig(g[:, 0:2 * _H])
            sb = _sig(g[:, 2 * _H:4 * _H])
            cpair = jnp.concatenate([lch[:, 0:_H], rch[:, 0:_H]], axis=1)
            fc = sb * cpair
            cg = (sa[:, 0:_H] * jnp.tanh(g[:, 4 * _H:5 * _H])
                  + fc[:, 0:_H] + fc[:, _H:])
        hg = sa[:, _H:2 * _H] * jnp.tanh(cg)
        sf_ref[s:e, :] = jnp.concatenate([cg, hg], axis=1)
        dst_ref[s:e, 0:_H] = hg

    def bwd_level(l, x_ref, s, e):
        wx = wxb_ref[l]
        wh = whb_ref[l]
        m = e - s
        if s == 0:
            g = jnp.dot(x_ref[0:1, :], wx, preferred_element_type=f32)
            g = g + bbb_ref[l]
            sa = _sig(g[:, 0:2 * _H])
            cg = sa[:, 0:_H] * jnp.tanh(g[:, 3 * _H:4 * _H])
            hg = sa[:, _H:2 * _H] * jnp.tanh(cg)
            sb_ref[0:1, :] = jnp.concatenate([cg, hg], axis=1)
            return
        ps = (s - 1) // 2
        pe = (e - 2) // 2 + 1
        pch = sb_ref[ps:pe, :]
        pg = jnp.dot(pch, wh, preferred_element_type=f32)
        pg = pg + bbb_ref[l]
        for par, n_p in ((0, (m + 1) // 2), (1, m // 2)):
            xs = x_ref[pl.Slice(s + par, n_p, 2), :]
            g = jnp.dot(xs, wx, preferred_element_type=f32)
            g = g + pg[0:n_p, :]
            sa = _sig(g[:, 0:2 * _H])
            t = jnp.tanh(jnp.concatenate(
                [0.5 * g[:, 2 * _H:3 * _H], g[:, 3 * _H:4 * _H]], axis=1))
            cg = (sa[:, 0:_H] * t[:, _H:2 * _H]
                  + (0.5 * t[:, 0:_H] + 0.5) * pch[0:n_p, 0:_H])
            hg = sa[:, _H:2 * _H] * jnp.tanh(cg)
            sb_ref[pl.Slice(s + par, n_p, 2), :] = (
                jnp.concatenate([cg, hg], axis=1))

    def layer(l, x_ref, dst_ref):
        nst = max(len(_FWD_CHUNKS), len(_LEVELS))
        for k in range(nst):
            if k < len(_FWD_CHUNKS):
                fwd_chunk(l, x_ref, dst_ref, *_FWD_CHUNKS[k])
            if k < len(_LEVELS):
                bwd_level(l, x_ref, *_LEVELS[k])
        dst_ref[0:_N, _H:2 * _H] = sb_ref[0:_N, _H:2 * _H]

    layer(0, feat_ref, x1_ref)
    layer(1, x1_ref, out_ref)


def kernel(features, Wxf, bxf, Wlf, blf, Wrf, brf, Wxb, bxb, Whb, bhb):
    f32 = jnp.float32
    pad_g = ((0, 0), (0, 0), (0, _H))
    pad_c = ((0, 0), (_H, 0), (0, _H))
    wxfT = jnp.pad(jnp.transpose(Wxf, (0, 2, 1)), pad_g)
    wlfT = jnp.pad(jnp.transpose(Wlf, (0, 2, 1)), pad_c)
    wrfT = jnp.pad(jnp.transpose(Wrf, (0, 2, 1)), pad_c)
    wall = jnp.concatenate([wxfT, wlfT, wrfT], axis=1)
    wxbT = jnp.transpose(Wxb, (0, 2, 1))
    whbT = jnp.pad(jnp.transpose(Whb, (0, 2, 1)),
                   ((0, 0), (_H, 0), (0, 0)))
    pad_b = ((0, 0), (0, _H))
    bff = jnp.pad(bxf + blf + brf, pad_b)[:, None, :]
    bxl = jnp.pad(bxf + blf, pad_b)[:, None, :]
    bbb = (bxb + bhb)[:, None, :]
    wnone = jnp.pad(jnp.concatenate(
        [wxfT[:, :, 0:2 * _H], wxfT[:, :, 4 * _H:5 * _H]], axis=2),
        ((0, 0), (0, 0), (0, _H)))
    bnone = jnp.pad(jnp.concatenate(
        [bxf[:, 0:2 * _H], bxf[:, 4 * _H:5 * _H]], axis=1),
        ((0, 0), (0, _H)))[:, None, :]
    return pl.pallas_call(
        _btree_kernel,
        out_shape=jax.ShapeDtypeStruct((_N, _D), f32),
        scratch_shapes=[
            pltpu.VMEM((_N, _D), f32),
            pltpu.VMEM((_N, 2 * _H), f32),
            pltpu.VMEM((_N, 2 * _H), f32),
        ],
    )(features.astype(f32), wall, wnone, bnone, bff, bxl,
      wxbT, whbT, bbb)


# speedup vs baseline: 1.0390x; 1.0390x over previous
import jax
import jax.numpy as jnp
from jax.experimental import pallas as pl
from jax.experimental.pallas import tpu as pltpu

_N = 10000
_D = 128
_H = 64
_L = 2


def _level_bounds(n):
    levs = []
    start, size = 0, 1
    while start < n:
        levs.append((start, min(start + size, n)))
        start += size
        size *= 2
    return levs


_LEVELS = _level_bounds(_N)

_FWD_CHUNKS = []
for _s, _e in reversed(_LEVELS):
    _nl = min(max((_N - 2) // 2 + 1 - _s, 0), _e - _s)
    _nr = min(max((_N - 3) // 2 + 1 - _s, 0), _e - _s)
    if _nr > 0:
        _FWD_CHUNKS.append((_s, _s + _nr, 'both'))
    if _nl > _nr:
        _FWD_CHUNKS.append((_s + _nr, _s + _nl, 'left'))
    if _nl < _e - _s:
        _FWD_CHUNKS.append((_s + _nl, _e, 'none'))


def _sig(v):
    return 0.5 * jnp.tanh(0.5 * v) + 0.5


def _btree_kernel(feat_ref, wall_ref, bxf_ref, bff_ref, blf_ref,
                  wxb_ref, whb_ref, bbb_ref, out_ref,
                  x1_ref, sf_ref, sb_ref):
    f32 = jnp.float32

    def fwd_chunk(l, x_ref, dst_ref, s, e, kind):
        m = e - s
        xs = x_ref[s:e, :]
        if kind == 'none':
            g = jnp.dot(xs, wall_ref[l, 0:_D, :], preferred_element_type=f32)
            g = g + bxf_ref[l]
            sa = _sig(g[:, 0:2 * _H])
            cg = sa[:, 0:_H] * jnp.tanh(g[:, 4 * _H:5 * _H])
        elif kind == 'left':
            lch = sf_ref[pl.Slice(2 * s + 1, m, 2), :]
            g = jnp.dot(jnp.concatenate([xs, lch], axis=1),
                        wall_ref[l, 0:_D + 2 * _H, :],
                        preferred_element_type=f32)
            g = g + blf_ref[l]
            sa = _sig(g[:, 0:2 * _H])
            sb = _sig(g[:, 2 * _H:4 * _H])
            cg = (sa[:, 0:_H] * jnp.tanh(g[:, 4 * _H:5 * _H])
                  + sb[:, 0:_H] * lch[:, 0:_H])
        else:
            cs = 2 * s + 1
            lch = sf_ref[pl.Slice(cs, m, 2), :]
            rch = sf_ref[pl.Slice(cs + 1, m, 2), :]
            g = jnp.dot(jnp.concatenate([xs, lch, rch], axis=1), wall_ref[l],
                        preferred_element_type=f32)
            g = g + bff_ref[l]
            sa = _sig(g[:, 0:2 * _H])
            sb = _sig(g[:, 2 * _H:4 * _H])
            cpair = jnp.concatenate([lch[:, 0:_H], rch[:, 0:_H]], axis=1)
            fc = sb * cpair
            cg = (sa[:, 0:_H] * jnp.tanh(g[:, 4 * _H:5 * _H])
                  + fc[:, 0:_H] + fc[:, _H:])
        hg = sa[:, _H:2 * _H] * jnp.tanh(cg)
        sf_ref[s:e, :] = jnp.concatenate([cg, hg], axis=1)
        dst_ref[s:e, 0:_H] = hg

    def bwd_level(l, x_ref, s, e):
        wx = wxb_ref[l]
        wh = whb_ref[l]
        m = e - s
        if s == 0:
            g = jnp.dot(x_ref[0:1, :], wx, preferred_element_type=f32)
            g = g + bbb_ref[l]
            sa = _sig(g[:, 0:2 * _H])
            cg = sa[:, 0:_H] * jnp.tanh(g[:, 3 * _H:4 * _H])
            hg = sa[:, _H:2 * _H] * jnp.tanh(cg)
            sb_ref[0:1, :] = jnp.concatenate([cg, hg], axis=1)
            return
        ps = (s - 1) // 2
        pe = (e - 2) // 2 + 1
        pch = sb_ref[ps:pe, :]
        pg = jnp.dot(pch, wh, preferred_element_type=f32)
        pg = pg + bbb_ref[l]
        for par, n_p in ((0, (m + 1) // 2), (1, m // 2)):
            xs = x_ref[pl.Slice(s + par, n_p, 2), :]
            g = jnp.dot(xs, wx, preferred_element_type=f32)
            g = g + pg[0:n_p, :]
            sa = _sig(g[:, 0:2 * _H])
            t = jnp.tanh(jnp.concatenate(
                [0.5 * g[:, 2 * _H:3 * _H], g[:, 3 * _H:4 * _H]], axis=1))
            cg = (sa[:, 0:_H] * t[:, _H:2 * _H]
                  + (0.5 * t[:, 0:_H] + 0.5) * pch[0:n_p, 0:_H])
            hg = sa[:, _H:2 * _H] * jnp.tanh(cg)
            sb_ref[pl.Slice(s + par, n_p, 2), :] = (
                jnp.concatenate([cg, hg], axis=1))

    def layer(l, x_ref, dst_ref):
        nst = max(len(_FWD_CHUNKS), len(_LEVELS))
        for k in range(nst):
            if k < len(_FWD_CHUNKS):
                fwd_chunk(l, x_ref, dst_ref, *_FWD_CHUNKS[k])
            if k < len(_LEVELS):
                bwd_level(l, x_ref, *_LEVELS[k])
        dst_ref[0:_N, _H:2 * _H] = sb_ref[0:_N, _H:2 * _H]

    layer(0, feat_ref, x1_ref)
    layer(1, x1_ref, out_ref)


def kernel(features, Wxf, bxf, Wlf, blf, Wrf, brf, Wxb, bxb, Whb, bhb):
    f32 = jnp.float32
    pad_g = ((0, 0), (0, 0), (0, _H))
    pad_c = ((0, 0), (_H, 0), (0, _H))
    wxfT = jnp.pad(jnp.transpose(Wxf, (0, 2, 1)), pad_g)
    wlfT = jnp.pad(jnp.transpose(Wlf, (0, 2, 1)), pad_c)
    wrfT = jnp.pad(jnp.transpose(Wrf, (0, 2, 1)), pad_c)
    wall = jnp.concatenate([wxfT, wlfT, wrfT], axis=1)
    wxbT = jnp.transpose(Wxb, (0, 2, 1))
    whbT = jnp.pad(jnp.transpose(Whb, (0, 2, 1)),
                   ((0, 0), (_H, 0), (0, 0)))
    pad_b = ((0, 0), (0, _H))
    bxf1 = jnp.pad(bxf, pad_b)[:, None, :]
    bff = jnp.pad(bxf + blf + brf, pad_b)[:, None, :]
    bxl = jnp.pad(bxf + blf, pad_b)[:, None, :]
    bbb = (bxb + bhb)[:, None, :]
    return pl.pallas_call(
        _btree_kernel,
        out_shape=jax.ShapeDtypeStruct((_N, _D), f32),
        scratch_shapes=[
            pltpu.VMEM((_N, _D), f32),
            pltpu.VMEM((_N, 2 * _H), f32),
            pltpu.VMEM((_N, 2 * _H), f32),
        ],
    )(features.astype(f32), wall, bxf1, bff, bxl,
      wxbT, whbT, bbb)
